# Initial kernel scaffold; baseline (speedup 1.0000x reference)
import jax
import jax.numpy as jnp
from jax.experimental import pallas as pl


def kernel(x, w):
    raise NotImplementedError("write your pallas kernel here")


# speedup vs baseline: 1.0580x; 1.0580x over previous
import functools
import jax
import jax.numpy as jnp
from jax.experimental import pallas as pl
from jax.experimental.pallas import tpu as pltpu

_C = 384
_HW = 3136


def _body(x_ref, w_ref, out_ref):
    xv = x_ref[0]
    y = jnp.sum(xv, axis=1) * (1.0 / _HW)
    yr = y.reshape(1, _C)
    iota = jax.lax.broadcasted_iota(jnp.int32, (1, _C), 1)
    w0 = w_ref[0]
    w1 = w_ref[1]
    w2 = w_ref[2]
    yprev = jnp.where(iota == 0, 0.0, pltpu.roll(yr, 1, axis=1))
    ynext = jnp.where(iota == _C - 1, 0.0, pltpu.roll(yr, _C - 1, axis=1))
    s = w0 * yprev + w1 * yr + w2 * ynext
    cur = s
    for k in range(3):
        m = jnp.max(cur)
        idx_k = jnp.min(jnp.where(cur == m, iota, _C))
        row = x_ref[0, pl.ds(idx_k, 1), :]
        out_ref[0, pl.ds(k, 1), :] = row
        cur = jnp.where(iota == idx_k, -jnp.inf, cur)


@jax.jit
def kernel(x, w):
    b, c, h, wd = x.shape
    x3 = x.reshape(b, c, h * wd)
    out = pl.pallas_call(
        _body,
        grid=(b,),
        in_specs=[
            pl.BlockSpec((1, c, h * wd), lambda i: (i, 0, 0)),
            pl.BlockSpec(memory_space=pltpu.SMEM),
        ],
        out_specs=pl.BlockSpec((1, 3, h * wd), lambda i: (i, 0, 0)),
        out_shape=jax.ShapeDtypeStruct((b, 3, h * wd), x.dtype),
    )(x3, w)
    return out.reshape(b, 3, h, wd)
